# Initial kernel scaffold; baseline (speedup 1.0000x reference)
import jax
import jax.numpy as jnp
from jax.experimental import pallas as pl


def kernel(input):
    raise NotImplementedError("write your pallas kernel here")


# speedup vs baseline: 18.5852x; 18.5852x over previous
import functools

import jax
import jax.numpy as jnp
from jax.experimental import pallas as pl


def _twd_block(x_ref, o_ref, *, k):
    a = jnp.abs(x_ref[...])
    key = jax.lax.bitcast_convert_type(a, jnp.int32)
    rows = a.shape[0]
    acc = jnp.zeros((rows, 1), jnp.int32)
    for bit in range(30, -1, -1):
        trial = acc | (1 << bit)
        cnt = jnp.sum((key < trial).astype(jnp.int32), axis=1, keepdims=True)
        acc = jnp.where(cnt <= k, trial, acc)
    o_ref[...] = jnp.where(key > acc, a, 0.0)


def kernel(input):
    nrows, ncols = input.shape
    k = int(0.5 * ncols)
    block_rows = 256
    return pl.pallas_call(
        functools.partial(_twd_block, k=k),
        grid=(nrows // block_rows,),
        in_specs=[pl.BlockSpec((block_rows, ncols), lambda i: (i, 0))],
        out_specs=pl.BlockSpec((block_rows, ncols), lambda i: (i, 0)),
        out_shape=jax.ShapeDtypeStruct((nrows, ncols), jnp.float32),
    )(input)
